# baseline (speedup 1.0000x reference)
import jax
import jax.numpy as jnp
from jax.experimental import pallas as pl
from jax.experimental.pallas import tpu as pltpu

_N = 2_000_000
_COLS = 128
_ROWS = _N // _COLS
_CH = 1000
_NCH = -(-_ROWS // _CH)
_LAST = _ROWS - (_NCH - 1) * _CH
_NBUF = 3


def _rows(j):
    return _LAST if j == _NCH - 1 else _CH


def _tc_body(s_ref, t_ref, x_hbm, o_hbm, xb, ob, si, so):
    s = s_ref[0, 0]
    t = t_ref[0, 0]

    def in_copy(j):
        return pltpu.make_async_copy(
            x_hbm.at[pl.ds(j * _CH, _rows(j))],
            xb.at[j % _NBUF, pl.ds(0, _rows(j))],
            si.at[j % _NBUF],
        )

    def out_copy(j):
        return pltpu.make_async_copy(
            ob.at[j % _NBUF, pl.ds(0, _rows(j))],
            o_hbm.at[pl.ds(j * _CH, _rows(j))],
            so.at[j % _NBUF],
        )

    for j in range(min(_NBUF, _NCH)):
        in_copy(j).start()
    for j in range(_NCH):
        in_copy(j).wait()
        if j >= _NBUF:
            out_copy(j - _NBUF).wait()
        r = _rows(j)
        ob[j % _NBUF, pl.ds(0, r)] = xb[j % _NBUF, pl.ds(0, r)] * s + t
        out_copy(j).start()
        if j + _NBUF < _NCH:
            in_copy(j + _NBUF).start()
    for j in range(max(_NCH - _NBUF, 0), _NCH):
        out_copy(j).wait()


@jax.jit
def _tc_affine(x2, s11, t11):
    return pl.pallas_call(
        _tc_body,
        in_specs=[
            pl.BlockSpec(memory_space=pltpu.SMEM),
            pl.BlockSpec(memory_space=pltpu.SMEM),
            pl.BlockSpec(memory_space=pl.ANY),
        ],
        out_specs=pl.BlockSpec(memory_space=pl.ANY),
        out_shape=jax.ShapeDtypeStruct((_ROWS, _COLS), jnp.float32),
        scratch_shapes=[
            pltpu.VMEM((_NBUF, _CH, _COLS), jnp.float32),
            pltpu.VMEM((_NBUF, _CH, _COLS), jnp.float32),
            pltpu.SemaphoreType.DMA((_NBUF,)),
            pltpu.SemaphoreType.DMA((_NBUF,)),
        ],
    )(s11, t11, x2)


def kernel(x, head, scale, shift):
    del head
    x2 = jnp.reshape(x, (_ROWS, _COLS))
    s11 = jnp.reshape(scale.astype(jnp.float32), (1, 1))
    t11 = jnp.reshape(shift.astype(jnp.float32), (1, 1))
    return jnp.reshape(_tc_affine(x2, s11, t11), (_N,))


# speedup vs baseline: 1.4018x; 1.4018x over previous
import jax
import jax.numpy as jnp
from jax.experimental import pallas as pl
from jax.experimental.pallas import tpu as pltpu

_N = 2_000_000
_COLS = 128
_ROWS = _N // _COLS
_CH = 4000
_NCH = -(-_ROWS // _CH)
_LAST = _ROWS - (_NCH - 1) * _CH
_NBUF = 3


def _rows(j):
    return _LAST if j == _NCH - 1 else _CH


def _tc_body(s_ref, t_ref, x_hbm, o_hbm, xb, ob, si, so):
    s = s_ref[0, 0]
    t = t_ref[0, 0]

    def in_copy(j):
        return pltpu.make_async_copy(
            x_hbm.at[pl.ds(j * _CH, _rows(j))],
            xb.at[j % _NBUF, pl.ds(0, _rows(j))],
            si.at[j % _NBUF],
        )

    def out_copy(j):
        return pltpu.make_async_copy(
            ob.at[j % _NBUF, pl.ds(0, _rows(j))],
            o_hbm.at[pl.ds(j * _CH, _rows(j))],
            so.at[j % _NBUF],
        )

    for j in range(min(_NBUF, _NCH)):
        in_copy(j).start()
    for j in range(_NCH):
        in_copy(j).wait()
        if j >= _NBUF:
            out_copy(j - _NBUF).wait()
        r = _rows(j)
        ob[j % _NBUF, pl.ds(0, r)] = xb[j % _NBUF, pl.ds(0, r)] * s + t
        out_copy(j).start()
        if j + _NBUF < _NCH:
            in_copy(j + _NBUF).start()
    for j in range(max(_NCH - _NBUF, 0), _NCH):
        out_copy(j).wait()


@jax.jit
def _tc_affine(x2, s11, t11):
    return pl.pallas_call(
        _tc_body,
        in_specs=[
            pl.BlockSpec(memory_space=pltpu.SMEM),
            pl.BlockSpec(memory_space=pltpu.SMEM),
            pl.BlockSpec(memory_space=pl.ANY),
        ],
        out_specs=pl.BlockSpec(memory_space=pl.ANY),
        out_shape=jax.ShapeDtypeStruct((_ROWS, _COLS), jnp.float32),
        scratch_shapes=[
            pltpu.VMEM((_NBUF, _CH, _COLS), jnp.float32),
            pltpu.VMEM((_NBUF, _CH, _COLS), jnp.float32),
            pltpu.SemaphoreType.DMA((_NBUF,)),
            pltpu.SemaphoreType.DMA((_NBUF,)),
        ],
    )(s11, t11, x2)


def kernel(x, head, scale, shift):
    del head
    x2 = jnp.reshape(x, (_ROWS, _COLS))
    s11 = jnp.reshape(scale.astype(jnp.float32), (1, 1))
    t11 = jnp.reshape(shift.astype(jnp.float32), (1, 1))
    return jnp.reshape(_tc_affine(x2, s11, t11), (_N,))


# speedup vs baseline: 1.5074x; 1.0754x over previous
import jax
import jax.numpy as jnp
from jax.experimental import pallas as pl
from jax.experimental.pallas import tpu as pltpu

_N = 2_000_000
_COLS = 128
_ROWS = _N // _COLS
_CH = 8000
_NCH = -(-_ROWS // _CH)
_LAST = _ROWS - (_NCH - 1) * _CH
_NBUF = 2


def _rows(j):
    return _LAST if j == _NCH - 1 else _CH


def _tc_body(s_ref, t_ref, x_hbm, o_hbm, xb, ob, si, so):
    s = s_ref[0, 0]
    t = t_ref[0, 0]

    def in_copy(j):
        return pltpu.make_async_copy(
            x_hbm.at[pl.ds(j * _CH, _rows(j))],
            xb.at[j % _NBUF, pl.ds(0, _rows(j))],
            si.at[j % _NBUF],
        )

    def out_copy(j):
        return pltpu.make_async_copy(
            ob.at[j % _NBUF, pl.ds(0, _rows(j))],
            o_hbm.at[pl.ds(j * _CH, _rows(j))],
            so.at[j % _NBUF],
        )

    for j in range(min(_NBUF, _NCH)):
        in_copy(j).start()
    for j in range(_NCH):
        in_copy(j).wait()
        if j >= _NBUF:
            out_copy(j - _NBUF).wait()
        r = _rows(j)
        ob[j % _NBUF, pl.ds(0, r)] = xb[j % _NBUF, pl.ds(0, r)] * s + t
        out_copy(j).start()
        if j + _NBUF < _NCH:
            in_copy(j + _NBUF).start()
    for j in range(max(_NCH - _NBUF, 0), _NCH):
        out_copy(j).wait()


@jax.jit
def _tc_affine(x2, s11, t11):
    return pl.pallas_call(
        _tc_body,
        in_specs=[
            pl.BlockSpec(memory_space=pltpu.SMEM),
            pl.BlockSpec(memory_space=pltpu.SMEM),
            pl.BlockSpec(memory_space=pl.ANY),
        ],
        out_specs=pl.BlockSpec(memory_space=pl.ANY),
        out_shape=jax.ShapeDtypeStruct((_ROWS, _COLS), jnp.float32),
        scratch_shapes=[
            pltpu.VMEM((_NBUF, _CH, _COLS), jnp.float32),
            pltpu.VMEM((_NBUF, _CH, _COLS), jnp.float32),
            pltpu.SemaphoreType.DMA((_NBUF,)),
            pltpu.SemaphoreType.DMA((_NBUF,)),
        ],
    )(s11, t11, x2)


def kernel(x, head, scale, shift):
    del head
    x2 = jnp.reshape(x, (_ROWS, _COLS))
    s11 = jnp.reshape(scale.astype(jnp.float32), (1, 1))
    t11 = jnp.reshape(shift.astype(jnp.float32), (1, 1))
    return jnp.reshape(_tc_affine(x2, s11, t11), (_N,))
